# baseline (speedup 1.0000x reference)
import functools

import jax
import jax.numpy as jnp
import numpy as np
from jax.experimental import pallas as pl
from jax.experimental.pallas import tpu as pltpu

_NEG_INF = float("-inf")


def _fps_body(px_ref, py_ref, pz_ref, cx_ref, cy_ref, cz_ref, *, steps):
    B, R, L = px_ref.shape
    P = R * L
    px, py, pz = px_ref[...], py_ref[...], pz_ref[...]
    ii = (jax.lax.broadcasted_iota(jnp.int32, (B, R, L), 1) * L
          + jax.lax.broadcasted_iota(jnp.int32, (B, R, L), 2))

    W = min(L, steps)
    lane = jax.lax.broadcasted_iota(jnp.int32, (B, W), 1)

    def step(j, carry):
        dists, last, bx, by, bz = carry
        oh = ii == last
        lx = jnp.sum(jnp.where(oh, px, 0.0), axis=(1, 2), keepdims=True)
        ly = jnp.sum(jnp.where(oh, py, 0.0), axis=(1, 2), keepdims=True)
        lz = jnp.sum(jnp.where(oh, pz, 0.0), axis=(1, 2), keepdims=True)
        dx, dy, dz = px - lx, py - ly, pz - lz
        d = dx * dx + dy * dy + dz * dz
        dists = jnp.minimum(dists, d)
        m = jnp.max(dists, axis=(1, 2), keepdims=True)
        sel = jnp.where(dists == m, ii, P)
        nxt = jnp.min(sel, axis=(1, 2), keepdims=True)
        hit = lane == j
        bx = jnp.where(hit, lx[:, :, 0], bx)
        by = jnp.where(hit, ly[:, :, 0], by)
        bz = jnp.where(hit, lz[:, :, 0], bz)
        return dists, nxt, bx, by, bz

    dists = jnp.full((B, R, L), jnp.inf, jnp.float32)
    last = jnp.zeros((B, 1, 1), jnp.int32)
    zb = jnp.zeros((B, W), jnp.float32)
    for o in range(steps // W):
        dists, last, bx, by, bz = jax.lax.fori_loop(
            0, W, step, (dists, last, zb, zb, zb))
        cx_ref[:, o * W:(o + 1) * W] = bx
        cy_ref[:, o * W:(o + 1) * W] = by
        cz_ref[:, o * W:(o + 1) * W] = bz


def _fps(px, py, pz, steps, interpret=False):
    B, P = px.shape
    L = 128
    shp = (B, P // L, L)
    out = jax.ShapeDtypeStruct((B, steps), jnp.float32)
    return pl.pallas_call(
        functools.partial(_fps_body, steps=steps),
        out_shape=(out, out, out),
        interpret=interpret,
    )(px.reshape(shp), py.reshape(shp), pz.reshape(shp))


def _lin1_body(x_ref, p_ref, wa_ref, wb_ref, b_ref, o_ref):
    acc = jnp.dot(x_ref[...], wa_ref[...], preferred_element_type=jnp.float32)
    p = p_ref[...]
    for i in range(3):
        acc = acc + p[:, i:i + 1] * wb_ref[i:i + 1, :]
    o_ref[...] = acc + b_ref[...]


def _lin1(x, p, wa, wb, b, interpret=False):
    N = x.shape[0]
    D1 = wa.shape[1]
    return pl.pallas_call(
        _lin1_body,
        out_shape=jax.ShapeDtypeStruct((N, D1), jnp.float32),
        interpret=interpret,
    )(x, p, wa, wb, b.reshape(1, D1))


def _sa_body(cpos_ref, ppt_ref, u_ref, wb_ref, w2_ref, b2_ref, w3_ref,
             b3_ref, g3_ref, be3_ref, o_ref, d2s_ref, *,
             k, r2_bits, pc, n_iter):
    C = cpos_ref[0]
    Pt = ppt_ref[0]
    Sb = C.shape[0]
    P = Pt.shape[1]
    dx = C[:, 0:1] - Pt[0:1, :]
    dy = C[:, 1:2] - Pt[1:2, :]
    dz = C[:, 2:3] - Pt[2:3, :]
    d2 = dx * dx + dy * dy + dz * dz
    d2i = jax.lax.bitcast_convert_type(d2, jnp.int32)
    d2s_ref[...] = d2i

    def bisect(_, carry):
        lo, hi = carry
        mid = (lo + hi) >> 1
        cnt = jnp.sum(jnp.where(d2i <= mid, 1.0, 0.0), axis=1, keepdims=True)
        good = cnt <= k
        return jnp.where(good, mid, lo), jnp.where(good, hi, mid)

    lo0 = jnp.full((Sb, 1), -1, jnp.int32)
    hi0 = jnp.full((Sb, 1), r2_bits + 1, jnp.int32)
    lo, _ = jax.lax.fori_loop(0, n_iter, bisect, (lo0, hi0))
    mask = d2i <= lo
    count = jnp.sum(jnp.where(mask, 1.0, 0.0), axis=1, keepdims=True)

    H = jnp.dot(C, wb_ref[...], preferred_element_type=jnp.float32)
    w2, b2 = w2_ref[...], b2_ref[...]
    w3, b3 = w3_ref[...], b3_ref[...]
    g3, be3 = g3_ref[...], be3_ref[...]
    D3 = w3.shape[1]

    def chunk(c, acc):
        base = pl.multiple_of(c * pc, pc)
        uc = u_ref[0, pl.ds(base, pc), :]
        z1 = uc[:, None, :] - H[None, :, :]
        h1 = jnp.maximum(z1, 0.0).reshape(pc * Sb, -1).astype(jnp.bfloat16)
        h2 = jnp.maximum(
            jnp.dot(h1, w2, preferred_element_type=jnp.float32) + b2,
            0.0).astype(jnp.bfloat16)
        h3 = jnp.maximum(
            jnp.dot(h2, w3, preferred_element_type=jnp.float32) + b3, 0.0)
        h3 = h3 * g3 + be3
        mc = d2s_ref[:, pl.ds(base, pc)] <= lo
        pen = jnp.where(mc, 0.0, -1e30)
        for j in range(pc):
            hv = h3[j * Sb:(j + 1) * Sb, :] + pen[:, j:j + 1]
            acc = jnp.maximum(acc, hv)
        return acc

    acc0 = jnp.full((Sb, D3), -1e30, jnp.float32)
    acc = jax.lax.fori_loop(0, P // pc, chunk, acc0)
    o_ref[0] = jnp.where(count > 0, acc, 0.0)


def _sa_dense(cpos, ppos_t, u, params, r, k, sb, pc, interpret=False):
    (w1, b1, g1, be1), (w2, b2, g2, be2), (w3, b3, g3, be3) = params
    F = w1.shape[0] - 3
    wb = w1[F:, :]
    w2f = (g1[:, None] * w2).astype(jnp.bfloat16)
    b2f = b2 + be1 @ w2
    w3f = (g2[:, None] * w3).astype(jnp.bfloat16)
    b3f = b3 + be2 @ w3
    B, S, _ = cpos.shape
    P = ppos_t.shape[2]
    D1, D2 = w2.shape
    D3 = w3.shape[1]
    r2_bits = int(np.float32(r * r).view(np.int32))
    grid = (B, S // sb)
    return pl.pallas_call(
        functools.partial(_sa_body, k=k, r2_bits=r2_bits, pc=pc, n_iter=31),
        grid=grid,
        in_specs=[
            pl.BlockSpec((1, sb, 3), lambda b, s: (b, s, 0)),
            pl.BlockSpec((1, 3, P), lambda b, s: (b, 0, 0)),
            pl.BlockSpec((1, P, D1), lambda b, s: (b, 0, 0)),
            pl.BlockSpec((3, D1), lambda b, s: (0, 0)),
            pl.BlockSpec((D1, D2), lambda b, s: (0, 0)),
            pl.BlockSpec((1, D2), lambda b, s: (0, 0)),
            pl.BlockSpec((D2, D3), lambda b, s: (0, 0)),
            pl.BlockSpec((1, D3), lambda b, s: (0, 0)),
            pl.BlockSpec((1, D3), lambda b, s: (0, 0)),
            pl.BlockSpec((1, D3), lambda b, s: (0, 0)),
        ],
        out_specs=pl.BlockSpec((1, sb, D3), lambda b, s: (b, s, 0)),
        out_shape=jax.ShapeDtypeStruct((B, S, D3), jnp.float32),
        scratch_shapes=[pltpu.VMEM((sb, P), jnp.int32)],
        interpret=interpret,
    )(cpos, ppos_t, u, wb, w2f, b2f.reshape(1, D2), w3f,
      b3f.reshape(1, D3), g3.reshape(1, D3), be3.reshape(1, D3))


def _pointnet2(x_0, pos_0, x_1, pos_1, p1, p2, *, k, r1, r2, interpret=False):
    B = 2
    P = x_0.shape[0] // B
    F = x_0.shape[-1]
    S1, S2 = P // 2, P // 8
    d1a = p1[0][0].shape[1]
    d1b = p2[0][0].shape[1]

    pos0 = pos_0.reshape(B, P, 3)
    pos1 = pos_1.reshape(B, P, 3)

    cx0, cy0, cz0 = _fps(pos0[:, :, 0], pos0[:, :, 1], pos0[:, :, 2], S1,
                         interpret)
    cx1, cy1, cz1 = _fps(cx0, cy0, cz0, S2, interpret)
    c0 = jnp.stack([cx0, cy0, cz0], axis=-1)
    c0_t = jnp.stack([cx0, cy0, cz0], axis=1)
    c1 = jnp.stack([cx1, cy1, cz1], axis=-1)

    pos0_t = jnp.transpose(pos0, (0, 2, 1))
    pos1_t = jnp.transpose(pos1, (0, 2, 1))

    sb1 = min(128, S1)
    sb2 = min(128, S2)
    pc1 = min(128, P)
    pc2 = min(128, S1)

    u1 = _lin1(x_0, pos_0, p1[0][0][:F], p1[0][0][F:], p1[0][1], interpret)
    x1_0 = _sa_dense(c0, pos0_t, u1.reshape(B, P, d1a), p1, r1, k,
                     sb1, pc1, interpret)

    u3 = _lin1(x_1, pos_1, p1[0][0][:F], p1[0][0][F:], p1[0][1], interpret)
    x1_1 = _sa_dense(c0, pos1_t, u3.reshape(B, P, d1a), p1, r1, k,
                     sb1, pc1, interpret)

    Fm = x1_0.shape[-1]
    u2 = _lin1(x1_0.reshape(B * S1, Fm), c0.reshape(B * S1, 3),
               p2[0][0][:Fm], p2[0][0][Fm:], p2[0][1], interpret)
    x2_0 = _sa_dense(c1, c0_t, u2.reshape(B, S1, d1b), p2, r2, k,
                     min(64, S2), pc2, interpret)

    u4 = _lin1(x1_1.reshape(B * S1, Fm), c0.reshape(B * S1, 3),
               p1[0][0][:Fm], p1[0][0][Fm:], p1[0][1], interpret)
    x2_1 = _sa_dense(c1, c0_t, u4.reshape(B, S1, d1a), p1, r1, k,
                     sb2, pc2, interpret)

    return x2_0, c1, x2_1


def kernel(x_0, pos_0, batch_0, x_1, pos_1, batch_1, p1, p2):
    B = 2
    P = x_0.shape[0] // B
    S2 = P // 8
    x2_0, c1, x2_1 = _pointnet2(x_0, pos_0, x_1, pos_1, p1, p2,
                                k=64, r1=0.2, r2=0.4)
    batch_c1 = batch_0.reshape(B, P)[:, :S2].reshape(-1)
    sa2_out_0 = (x2_0.reshape(-1, x2_0.shape[-1]), c1.reshape(-1, 3), batch_c1)
    sa2_out_1 = (x2_1.reshape(-1, x2_1.shape[-1]), c1.reshape(-1, 3), batch_c1)
    return (sa2_out_0, sa2_out_1)


# speedup vs baseline: 1.9706x; 1.9706x over previous
import functools

import jax
import jax.numpy as jnp
import numpy as np
from jax.experimental import pallas as pl
from jax.experimental.pallas import tpu as pltpu

_NEG_INF = float("-inf")


def _fps_body(px_ref, py_ref, pz_ref, cx_ref, cy_ref, cz_ref, *, steps):
    B, R, L = px_ref.shape
    P = R * L
    px, py, pz = px_ref[...], py_ref[...], pz_ref[...]
    ii = (jax.lax.broadcasted_iota(jnp.int32, (B, R, L), 1) * L
          + jax.lax.broadcasted_iota(jnp.int32, (B, R, L), 2))

    W = min(L, steps)
    lane = jax.lax.broadcasted_iota(jnp.int32, (B, W), 1)

    def step(j, carry):
        dists, last, bx, by, bz = carry
        oh = ii == last
        lx = jnp.sum(jnp.where(oh, px, 0.0), axis=(1, 2), keepdims=True)
        ly = jnp.sum(jnp.where(oh, py, 0.0), axis=(1, 2), keepdims=True)
        lz = jnp.sum(jnp.where(oh, pz, 0.0), axis=(1, 2), keepdims=True)
        dx, dy, dz = px - lx, py - ly, pz - lz
        d = dx * dx + dy * dy + dz * dz
        dists = jnp.minimum(dists, d)
        m = jnp.max(dists, axis=(1, 2), keepdims=True)
        sel = jnp.where(dists == m, ii, P)
        nxt = jnp.min(sel, axis=(1, 2), keepdims=True)
        hit = lane == j
        bx = jnp.where(hit, lx[:, :, 0], bx)
        by = jnp.where(hit, ly[:, :, 0], by)
        bz = jnp.where(hit, lz[:, :, 0], bz)
        return dists, nxt, bx, by, bz

    dists = jnp.full((B, R, L), jnp.inf, jnp.float32)
    last = jnp.zeros((B, 1, 1), jnp.int32)
    zb = jnp.zeros((B, W), jnp.float32)
    for o in range(steps // W):
        dists, last, bx, by, bz = jax.lax.fori_loop(
            0, W, step, (dists, last, zb, zb, zb))
        cx_ref[:, o * W:(o + 1) * W] = bx
        cy_ref[:, o * W:(o + 1) * W] = by
        cz_ref[:, o * W:(o + 1) * W] = bz


def _fps(px, py, pz, steps, interpret=False):
    B, P = px.shape
    L = 128
    shp = (B, P // L, L)
    out = jax.ShapeDtypeStruct((B, steps), jnp.float32)
    return pl.pallas_call(
        functools.partial(_fps_body, steps=steps),
        out_shape=(out, out, out),
        interpret=interpret,
    )(px.reshape(shp), py.reshape(shp), pz.reshape(shp))


def _lin1_body(x_ref, p_ref, wa_ref, wb_ref, b_ref, o_ref):
    acc = jnp.dot(x_ref[...], wa_ref[...], preferred_element_type=jnp.float32)
    p = p_ref[...]
    for i in range(3):
        acc = acc + p[:, i:i + 1] * wb_ref[i:i + 1, :]
    o_ref[...] = acc + b_ref[...]


def _lin1(x, p, wa, wb, b, interpret=False):
    N = x.shape[0]
    D1 = wa.shape[1]
    return pl.pallas_call(
        _lin1_body,
        out_shape=jax.ShapeDtypeStruct((N, D1), jnp.float32),
        interpret=interpret,
    )(x, p, wa, wb, b.reshape(1, D1))


def _sel_body(cpos_ref, ppt_ref, d2o_ref, lob_ref, cnt_ref, *,
              k, r2_bits, n_iter):
    C = cpos_ref[0]
    Pt = ppt_ref[0]
    Sb = C.shape[0]
    dx = C[:, 0:1] - Pt[0:1, :]
    dy = C[:, 1:2] - Pt[1:2, :]
    dz = C[:, 2:3] - Pt[2:3, :]
    d2 = dx * dx + dy * dy + dz * dz
    d2i = jax.lax.bitcast_convert_type(d2, jnp.int32)
    d2o_ref[0] = d2i

    def bisect(_, carry):
        lo, hi = carry
        mid = (lo + hi) >> 1
        cnt = jnp.sum(jnp.where(d2i <= mid, 1.0, 0.0), axis=1, keepdims=True)
        good = cnt <= k
        return jnp.where(good, mid, lo), jnp.where(good, hi, mid)

    lo0 = jnp.full((Sb, 1), -1, jnp.int32)
    hi0 = jnp.full((Sb, 1), r2_bits + 1, jnp.int32)
    lo, _ = jax.lax.fori_loop(0, n_iter, bisect, (lo0, hi0))
    count = jnp.sum(jnp.where(d2i <= lo, 1.0, 0.0), axis=1, keepdims=True)
    lob_ref[0] = jnp.broadcast_to(lo, (Sb, 16))
    cnt_ref[0] = count


def _sa_select(cpos, ppos_t, r, k, sb, interpret=False):
    B, S, _ = cpos.shape
    P = ppos_t.shape[2]
    r2_bits = int(np.float32(r * r).view(np.int32))
    return pl.pallas_call(
        functools.partial(_sel_body, k=k, r2_bits=r2_bits, n_iter=31),
        grid=(B, S // sb),
        in_specs=[
            pl.BlockSpec((1, sb, 3), lambda b, s: (b, s, 0)),
            pl.BlockSpec((1, 3, P), lambda b, s: (b, 0, 0)),
        ],
        out_specs=[
            pl.BlockSpec((1, sb, P), lambda b, s: (b, s, 0)),
            pl.BlockSpec((1, sb, 16), lambda b, s: (b, s, 0)),
            pl.BlockSpec((1, sb, 1), lambda b, s: (b, s, 0)),
        ],
        out_shape=[
            jax.ShapeDtypeStruct((B, S, P), jnp.int32),
            jax.ShapeDtypeStruct((B, S, 16), jnp.int32),
            jax.ShapeDtypeStruct((B, S, 1), jnp.float32),
        ],
        interpret=interpret,
    )(cpos, ppos_t)


def _sc_gather(d2_flat, lob, u, nrows, P, S, k, d1):
    from jax.experimental.pallas import tpu_sc as plsc
    info = plsc.get_sparse_core_info()
    NC, NS, L = info.num_cores, info.num_subcores, info.num_lanes
    NW = NC * NS
    rpt = nrows // NW
    g8 = min(8, rpt)
    mesh = plsc.VectorSubcoreMesh(core_axis_name="c", subcore_axis_name="s")

    @functools.partial(
        pl.kernel, mesh=mesh,
        compiler_params=pltpu.CompilerParams(
            use_tc_tiling_on_sc=False, needs_layout_passes=False),
        out_type=jax.ShapeDtypeStruct((nrows * k, d1), jnp.float32),
        scratch_types=[
            pltpu.VMEM((P,), jnp.int32),
            pltpu.VMEM((16,), jnp.int32),
            pltpu.VMEM((16,), jnp.int32),
            pltpu.VMEM((k,), jnp.int32),
            pltpu.VMEM((k, d1), jnp.float32),
            pltpu.SemaphoreType.DMA,
        ],
    )
    def body(d2_hbm, lob_hbm, u_hbm, gout_hbm, d2v, lov, stg, idxv, rowsv,
             sem):
        wid = jax.lax.axis_index("s") * NC + jax.lax.axis_index("c")
        row0 = wid * rpt
        iota = jax.lax.iota(jnp.int32, 16)
        zero16 = jnp.zeros((16,), jnp.int32)
        lane15 = jnp.full((16,), 15, jnp.int32)

        def do_row(rg, _):
            pltpu.sync_copy(d2_hbm.at[pl.ds(rg * P, P)], d2v)
            boff = (rg // S) * P
            pltpu.sync_copy(lob_hbm.at[rg], lov)
            lo_v = lov[...]
            for q in range(k // 16):
                idxv[q * 16:(q + 1) * 16] = zero16

            big = jnp.full((16,), jnp.int32(2**30), jnp.int32)

            def do_chunk(c, carry):
                base_v, fmin_v = carry
                off = pl.multiple_of(c * 16, 16)
                dv = d2v[pl.ds(off, 16)]
                m = dv <= lo_v
                mi = jnp.where(m, 1, 0)
                pos = plsc.cumsum(mi) + base_v - 1
                vals = boff + c * 16 + iota
                plsc.store_scatter(idxv, [pos], vals, mask=m)
                fmin_v = jnp.minimum(fmin_v, jnp.where(m, vals, big))
                return base_v + plsc.all_reduce_population_count(m), fmin_v

            base_v, fmin_v = jax.lax.fori_loop(
                0, P // 16, do_chunk, (zero16, big))
            first = jnp.minimum(jnp.min(fmin_v), boff + P - 1) + zero16
            for q in range(k // 16):
                cur = idxv[q * 16:(q + 1) * 16]
                slot = q * 16 + iota
                idxv[q * 16:(q + 1) * 16] = jnp.where(
                    slot < base_v, cur, first)
            pltpu.async_copy(u_hbm.at[idxv], rowsv, sem).wait()
            pltpu.sync_copy(
                rowsv, gout_hbm.at[pl.ds(pl.multiple_of(rg * k, k), k)])
            return 0

        jax.lax.fori_loop(0, rpt, lambda r, x: do_row(row0 + r, x), 0)

    return body(d2_flat, lob, u)


def _mlp_body(g_ref, cnt_ref, cpos_ref, wb_ref, w2_ref, b2_ref, w3_ref,
              b3_ref, g3_ref, be3_ref, o_ref, *, k):
    C = cpos_ref[0]
    Sb = C.shape[0]
    H = jnp.dot(C, wb_ref[...], preferred_element_type=jnp.float32)
    rows = g_ref[0]
    D1 = rows.shape[-1]
    z1 = rows.reshape(Sb, k, D1) - H[:, None, :]
    h1 = jnp.maximum(z1, 0.0).reshape(Sb * k, D1)
    h2 = jnp.maximum(
        jnp.dot(h1, w2_ref[...], preferred_element_type=jnp.float32)
        + b2_ref[...], 0.0)
    h3 = jnp.maximum(
        jnp.dot(h2, w3_ref[...], preferred_element_type=jnp.float32)
        + b3_ref[...], 0.0)
    h3 = h3 * g3_ref[...] + be3_ref[...]
    D3 = h3.shape[-1]
    hv = h3.reshape(Sb, k, D3)
    m = k
    while m > 1:
        m //= 2
        hv = jnp.maximum(hv[:, :m, :], hv[:, m:2 * m, :])
    acc = hv.reshape(Sb, D3)
    o_ref[0] = jnp.where(cnt_ref[0] > 0, acc, 0.0)


def _sa_mlp(gathered, counts, cpos, params, k, sb, interpret=False):
    (w1, b1, g1, be1), (w2, b2, g2, be2), (w3, b3, g3, be3) = params
    F = w1.shape[0] - 3
    wb = w1[F:, :]
    w2f = g1[:, None] * w2
    b2f = b2 + be1 @ w2
    w3f = g2[:, None] * w3
    b3f = b3 + be2 @ w3
    B, S, _ = cpos.shape
    D1, D2 = w2.shape
    D3 = w3.shape[1]
    g3d = gathered.reshape(B, S * k, D1)
    return pl.pallas_call(
        functools.partial(_mlp_body, k=k),
        grid=(B, S // sb),
        in_specs=[
            pl.BlockSpec((1, sb * k, D1), lambda b, s: (b, s, 0)),
            pl.BlockSpec((1, sb, 1), lambda b, s: (b, s, 0)),
            pl.BlockSpec((1, sb, 3), lambda b, s: (b, s, 0)),
            pl.BlockSpec((3, D1), lambda b, s: (0, 0)),
            pl.BlockSpec((D1, D2), lambda b, s: (0, 0)),
            pl.BlockSpec((1, D2), lambda b, s: (0, 0)),
            pl.BlockSpec((D2, D3), lambda b, s: (0, 0)),
            pl.BlockSpec((1, D3), lambda b, s: (0, 0)),
            pl.BlockSpec((1, D3), lambda b, s: (0, 0)),
            pl.BlockSpec((1, D3), lambda b, s: (0, 0)),
        ],
        out_specs=pl.BlockSpec((1, sb, D3), lambda b, s: (b, s, 0)),
        out_shape=jax.ShapeDtypeStruct((B, S, D3), jnp.float32),
        interpret=interpret,
    )(g3d, counts, cpos, wb, w2f, b2f.reshape(1, D2), w3f,
      b3f.reshape(1, D3), g3.reshape(1, D3), be3.reshape(1, D3))


def _sa_module_sc(cpos, ppos_t, u, params, r, k, sb, interpret=False):
    B, S, _ = cpos.shape
    P = ppos_t.shape[2]
    D1 = u.shape[-1]
    d2i, lob, counts = _sa_select(cpos, ppos_t, r, k, sb, interpret)
    nrows = B * S
    gathered = _sc_gather(d2i.reshape(nrows * P), lob.reshape(nrows, 16),
                          u.reshape(B * P, D1), nrows, P, S, k, D1)
    return _sa_mlp(gathered, counts, cpos, params, k, sb, interpret)


def _sa_body(cpos_ref, ppt_ref, u_ref, wb_ref, w2_ref, b2_ref, w3_ref,
             b3_ref, g3_ref, be3_ref, o_ref, d2s_ref, *,
             k, r2_bits, pc, n_iter):
    C = cpos_ref[0]
    Pt = ppt_ref[0]
    Sb = C.shape[0]
    P = Pt.shape[1]
    dx = C[:, 0:1] - Pt[0:1, :]
    dy = C[:, 1:2] - Pt[1:2, :]
    dz = C[:, 2:3] - Pt[2:3, :]
    d2 = dx * dx + dy * dy + dz * dz
    d2i = jax.lax.bitcast_convert_type(d2, jnp.int32)
    d2s_ref[...] = d2i

    def bisect(_, carry):
        lo, hi = carry
        mid = (lo + hi) >> 1
        cnt = jnp.sum(jnp.where(d2i <= mid, 1.0, 0.0), axis=1, keepdims=True)
        good = cnt <= k
        return jnp.where(good, mid, lo), jnp.where(good, hi, mid)

    lo0 = jnp.full((Sb, 1), -1, jnp.int32)
    hi0 = jnp.full((Sb, 1), r2_bits + 1, jnp.int32)
    lo, _ = jax.lax.fori_loop(0, n_iter, bisect, (lo0, hi0))
    mask = d2i <= lo
    count = jnp.sum(jnp.where(mask, 1.0, 0.0), axis=1, keepdims=True)

    H = jnp.dot(C, wb_ref[...], preferred_element_type=jnp.float32)
    w2, b2 = w2_ref[...], b2_ref[...]
    w3, b3 = w3_ref[...], b3_ref[...]
    g3, be3 = g3_ref[...], be3_ref[...]
    D3 = w3.shape[1]

    def chunk(c, acc):
        base = pl.multiple_of(c * pc, pc)
        uc = u_ref[0, pl.ds(base, pc), :]
        z1 = uc[:, None, :] - H[None, :, :]
        h1 = jnp.maximum(z1, 0.0).reshape(pc * Sb, -1)
        h2 = jnp.maximum(
            jnp.dot(h1, w2, preferred_element_type=jnp.float32) + b2, 0.0)
        h3 = jnp.maximum(
            jnp.dot(h2, w3, preferred_element_type=jnp.float32) + b3, 0.0)
        h3 = h3 * g3 + be3
        mc = d2s_ref[:, pl.ds(base, pc)] <= lo
        pen = jnp.where(mc, 0.0, -1e30)
        for j in range(pc):
            hv = h3[j * Sb:(j + 1) * Sb, :] + pen[:, j:j + 1]
            acc = jnp.maximum(acc, hv)
        return acc

    acc0 = jnp.full((Sb, D3), -1e30, jnp.float32)
    acc = jax.lax.fori_loop(0, P // pc, chunk, acc0)
    o_ref[0] = jnp.where(count > 0, acc, 0.0)


def _sa_dense(cpos, ppos_t, u, params, r, k, sb, pc, interpret=False):
    (w1, b1, g1, be1), (w2, b2, g2, be2), (w3, b3, g3, be3) = params
    F = w1.shape[0] - 3
    wb = w1[F:, :]
    w2f = g1[:, None] * w2
    b2f = b2 + be1 @ w2
    w3f = g2[:, None] * w3
    b3f = b3 + be2 @ w3
    B, S, _ = cpos.shape
    P = ppos_t.shape[2]
    D1, D2 = w2.shape
    D3 = w3.shape[1]
    r2_bits = int(np.float32(r * r).view(np.int32))
    grid = (B, S // sb)
    return pl.pallas_call(
        functools.partial(_sa_body, k=k, r2_bits=r2_bits, pc=pc, n_iter=31),
        grid=grid,
        in_specs=[
            pl.BlockSpec((1, sb, 3), lambda b, s: (b, s, 0)),
            pl.BlockSpec((1, 3, P), lambda b, s: (b, 0, 0)),
            pl.BlockSpec((1, P, D1), lambda b, s: (b, 0, 0)),
            pl.BlockSpec((3, D1), lambda b, s: (0, 0)),
            pl.BlockSpec((D1, D2), lambda b, s: (0, 0)),
            pl.BlockSpec((1, D2), lambda b, s: (0, 0)),
            pl.BlockSpec((D2, D3), lambda b, s: (0, 0)),
            pl.BlockSpec((1, D3), lambda b, s: (0, 0)),
            pl.BlockSpec((1, D3), lambda b, s: (0, 0)),
            pl.BlockSpec((1, D3), lambda b, s: (0, 0)),
        ],
        out_specs=pl.BlockSpec((1, sb, D3), lambda b, s: (b, s, 0)),
        out_shape=jax.ShapeDtypeStruct((B, S, D3), jnp.float32),
        scratch_shapes=[pltpu.VMEM((sb, P), jnp.int32)],
        interpret=interpret,
    )(cpos, ppos_t, u, wb, w2f, b2f.reshape(1, D2), w3f,
      b3f.reshape(1, D3), g3.reshape(1, D3), be3.reshape(1, D3))


def _pointnet2(x_0, pos_0, x_1, pos_1, p1, p2, *, k, r1, r2, interpret=False):
    B = 2
    P = x_0.shape[0] // B
    F = x_0.shape[-1]
    S1, S2 = P // 2, P // 8
    d1a = p1[0][0].shape[1]
    d1b = p2[0][0].shape[1]

    pos0 = pos_0.reshape(B, P, 3)
    pos1 = pos_1.reshape(B, P, 3)

    cx0, cy0, cz0 = _fps(pos0[:, :, 0], pos0[:, :, 1], pos0[:, :, 2], S1,
                         interpret)
    cx1, cy1, cz1 = _fps(cx0, cy0, cz0, S2, interpret)
    c0 = jnp.stack([cx0, cy0, cz0], axis=-1)
    c0_t = jnp.stack([cx0, cy0, cz0], axis=1)
    c1 = jnp.stack([cx1, cy1, cz1], axis=-1)

    pos0_t = jnp.transpose(pos0, (0, 2, 1))
    pos1_t = jnp.transpose(pos1, (0, 2, 1))

    sb1 = min(128, S1)
    sb2 = min(128, S2)
    pc1 = min(128, P)
    pc2 = min(128, S1)

    if interpret:
        def sa(cpos, ppos_t, u, params, r, sb, pc):
            return _sa_dense(cpos, ppos_t, u, params, r, k, sb, pc, True)
    else:
        def sa(cpos, ppos_t, u, params, r, sb, pc):
            return _sa_module_sc(cpos, ppos_t, u, params, r, k, sb)

    u1 = _lin1(x_0, pos_0, p1[0][0][:F], p1[0][0][F:], p1[0][1], interpret)
    x1_0 = sa(c0, pos0_t, u1.reshape(B, P, d1a), p1, r1, sb1, pc1)

    u3 = _lin1(x_1, pos_1, p1[0][0][:F], p1[0][0][F:], p1[0][1], interpret)
    x1_1 = sa(c0, pos1_t, u3.reshape(B, P, d1a), p1, r1, sb1, pc1)

    Fm = x1_0.shape[-1]
    u2 = _lin1(x1_0.reshape(B * S1, Fm), c0.reshape(B * S1, 3),
               p2[0][0][:Fm], p2[0][0][Fm:], p2[0][1], interpret)
    x2_0 = sa(c1, c0_t, u2.reshape(B, S1, d1b), p2, r2, min(64, S2), pc2)

    u4 = _lin1(x1_1.reshape(B * S1, Fm), c0.reshape(B * S1, 3),
               p1[0][0][:Fm], p1[0][0][Fm:], p1[0][1], interpret)
    x2_1 = sa(c1, c0_t, u4.reshape(B, S1, d1a), p1, r1, min(64, S2), pc2)

    return x2_0, c1, x2_1


def kernel(x_0, pos_0, batch_0, x_1, pos_1, batch_1, p1, p2):
    B = 2
    P = x_0.shape[0] // B
    S2 = P // 8
    x2_0, c1, x2_1 = _pointnet2(x_0, pos_0, x_1, pos_1, p1, p2,
                                k=64, r1=0.2, r2=0.4)
    batch_c1 = batch_0.reshape(B, P)[:, :S2].reshape(-1)
    sa2_out_0 = (x2_0.reshape(-1, x2_0.shape[-1]), c1.reshape(-1, 3), batch_c1)
    sa2_out_1 = (x2_1.reshape(-1, x2_1.shape[-1]), c1.reshape(-1, 3), batch_c1)
    return (sa2_out_0, sa2_out_1)


# speedup vs baseline: 2.4379x; 1.2372x over previous
import functools

import jax
import jax.numpy as jnp
import numpy as np
from jax.experimental import pallas as pl
from jax.experimental.pallas import tpu as pltpu

_NEG_INF = float("-inf")


def _fps_body(px_ref, py_ref, pz_ref, cx_ref, cy_ref, cz_ref, *, steps):
    B, R, L = px_ref.shape
    P = R * L
    px, py, pz = px_ref[...], py_ref[...], pz_ref[...]
    ii = (jax.lax.broadcasted_iota(jnp.int32, (B, R, L), 1) * L
          + jax.lax.broadcasted_iota(jnp.int32, (B, R, L), 2))

    W = min(L, steps)
    lane = jax.lax.broadcasted_iota(jnp.int32, (B, W), 1)

    def step(j, carry):
        dists, last, bx, by, bz = carry
        oh = ii == last
        lx = jnp.sum(jnp.where(oh, px, 0.0), axis=(1, 2), keepdims=True)
        ly = jnp.sum(jnp.where(oh, py, 0.0), axis=(1, 2), keepdims=True)
        lz = jnp.sum(jnp.where(oh, pz, 0.0), axis=(1, 2), keepdims=True)
        dx, dy, dz = px - lx, py - ly, pz - lz
        d = dx * dx + dy * dy + dz * dz
        dists = jnp.minimum(dists, d)
        m = jnp.max(dists, axis=(1, 2), keepdims=True)
        sel = jnp.where(dists == m, ii, P)
        nxt = jnp.min(sel, axis=(1, 2), keepdims=True)
        hit = lane == j
        bx = jnp.where(hit, lx[:, :, 0], bx)
        by = jnp.where(hit, ly[:, :, 0], by)
        bz = jnp.where(hit, lz[:, :, 0], bz)
        return dists, nxt, bx, by, bz

    dists = jnp.full((B, R, L), jnp.inf, jnp.float32)
    last = jnp.zeros((B, 1, 1), jnp.int32)
    zb = jnp.zeros((B, W), jnp.float32)
    for o in range(steps // W):
        dists, last, bx, by, bz = jax.lax.fori_loop(
            0, W, step, (dists, last, zb, zb, zb))
        cx_ref[:, o * W:(o + 1) * W] = bx
        cy_ref[:, o * W:(o + 1) * W] = by
        cz_ref[:, o * W:(o + 1) * W] = bz


def _fps(px, py, pz, steps, interpret=False):
    B, P = px.shape
    L = 128
    shp = (B, P // L, L)
    out = jax.ShapeDtypeStruct((B, steps), jnp.float32)
    return pl.pallas_call(
        functools.partial(_fps_body, steps=steps),
        out_shape=(out, out, out),
        interpret=interpret,
    )(px.reshape(shp), py.reshape(shp), pz.reshape(shp))


def _lin1_body(x_ref, p_ref, wa_ref, wb_ref, b_ref, o_ref):
    acc = jnp.dot(x_ref[...], wa_ref[...], preferred_element_type=jnp.float32)
    p = p_ref[...]
    for i in range(3):
        acc = acc + p[:, i:i + 1] * wb_ref[i:i + 1, :]
    o_ref[...] = acc + b_ref[...]


def _lin1(x, p, wa, wb, b, interpret=False):
    N = x.shape[0]
    D1 = wa.shape[1]
    return pl.pallas_call(
        _lin1_body,
        out_shape=jax.ShapeDtypeStruct((N, D1), jnp.float32),
        interpret=interpret,
    )(x, p, wa, wb, b.reshape(1, D1))


def _sel_body(cpos_ref, ppt_ref, d2o_ref, lob_ref, cnt_ref, *,
              k, r2_bits, n_iter):
    C = cpos_ref[0]
    Pt = ppt_ref[0]
    Sb = C.shape[0]
    dx = C[:, 0:1] - Pt[0:1, :]
    dy = C[:, 1:2] - Pt[1:2, :]
    dz = C[:, 2:3] - Pt[2:3, :]
    d2 = dx * dx + dy * dy + dz * dz
    d2i = jax.lax.bitcast_convert_type(d2, jnp.int32)
    d2o_ref[0] = d2i

    def bisect(_, carry):
        lo, hi = carry
        mid = (lo + hi) >> 1
        cnt = jnp.sum(jnp.where(d2i <= mid, 1.0, 0.0), axis=1, keepdims=True)
        good = cnt <= k
        return jnp.where(good, mid, lo), jnp.where(good, hi, mid)

    lo0 = jnp.full((Sb, 1), -1, jnp.int32)
    hi0 = jnp.full((Sb, 1), r2_bits + 1, jnp.int32)
    lo, _ = jax.lax.fori_loop(0, n_iter, bisect, (lo0, hi0))
    count = jnp.sum(jnp.where(d2i <= lo, 1.0, 0.0), axis=1, keepdims=True)
    lob_ref[0] = jnp.broadcast_to(lo, (Sb, 16))
    cnt_ref[0] = count


def _sa_select(cpos, ppos_t, r, k, sb, interpret=False):
    B, S, _ = cpos.shape
    P = ppos_t.shape[2]
    r2_bits = int(np.float32(r * r).view(np.int32))
    return pl.pallas_call(
        functools.partial(_sel_body, k=k, r2_bits=r2_bits, n_iter=31),
        grid=(B, S // sb),
        in_specs=[
            pl.BlockSpec((1, sb, 3), lambda b, s: (b, s, 0)),
            pl.BlockSpec((1, 3, P), lambda b, s: (b, 0, 0)),
        ],
        out_specs=[
            pl.BlockSpec((1, sb, P), lambda b, s: (b, s, 0)),
            pl.BlockSpec((1, sb, 16), lambda b, s: (b, s, 0)),
            pl.BlockSpec((1, sb, 1), lambda b, s: (b, s, 0)),
        ],
        out_shape=[
            jax.ShapeDtypeStruct((B, S, P), jnp.int32),
            jax.ShapeDtypeStruct((B, S, 16), jnp.int32),
            jax.ShapeDtypeStruct((B, S, 1), jnp.float32),
        ],
        interpret=interpret,
    )(cpos, ppos_t)


def _sc_gather(d2_flat, lob, u, nrows, P, S, k, d1):
    from jax.experimental.pallas import tpu_sc as plsc
    info = plsc.get_sparse_core_info()
    NC, NS, L = info.num_cores, info.num_subcores, info.num_lanes
    NW = NC * NS
    rpt = nrows // NW
    g8 = min(8, rpt)
    mesh = plsc.VectorSubcoreMesh(core_axis_name="c", subcore_axis_name="s")

    @functools.partial(
        pl.kernel, mesh=mesh,
        compiler_params=pltpu.CompilerParams(
            use_tc_tiling_on_sc=False, needs_layout_passes=False),
        out_type=jax.ShapeDtypeStruct((nrows * k, d1), jnp.float32),
        scratch_types=[
            pltpu.VMEM((g8 * P,), jnp.int32),
            pltpu.VMEM((g8 * 16,), jnp.int32),
            pltpu.VMEM((g8 * k,), jnp.int32),
            pltpu.VMEM((g8 * k, d1), jnp.float32),
            pltpu.SemaphoreType.DMA,
        ],
    )
    def body(d2_hbm, lob_hbm, u_hbm, gout_hbm, d2v, lovg, idxg, rowsg, sem):
        wid = jax.lax.axis_index("s") * NC + jax.lax.axis_index("c")
        row0 = wid * rpt
        iota = jax.lax.iota(jnp.int32, 16)
        zero16 = jnp.zeros((16,), jnp.int32)
        big = jnp.full((16,), jnp.int32(2**30), jnp.int32)

        def do_group(g, _):
            gbase = row0 + g * g8
            pltpu.sync_copy(d2_hbm.at[pl.ds(gbase * P, g8 * P)], d2v)
            pltpu.sync_copy(lob_hbm.at[pl.ds(gbase * 16, g8 * 16)], lovg)

            def do_row(rr, _):
                boff = ((gbase + rr) // S) * P
                lo_v = lovg[pl.ds(pl.multiple_of(rr * 16, 16), 16)]
                kbase = pl.multiple_of(rr * k, k)
                for q in range(k // 16):
                    idxg[pl.ds(kbase + q * 16, 16)] = zero16

                def do_chunk(c, carry):
                    base_v, fmin_v = carry
                    off = pl.multiple_of(rr * P + c * 16, 16)
                    dv = d2v[pl.ds(off, 16)]
                    m = dv <= lo_v
                    mi = jnp.where(m, 1, 0)
                    pos = plsc.cumsum(mi) + base_v - 1 + kbase
                    vals = boff + c * 16 + iota
                    plsc.store_scatter(idxg, [pos], vals, mask=m)
                    fmin_v = jnp.minimum(fmin_v, jnp.where(m, vals, big))
                    return (base_v + plsc.all_reduce_population_count(m),
                            fmin_v)

                base_v, fmin_v = jax.lax.fori_loop(
                    0, P // 16, do_chunk, (zero16, big))
                first = jnp.minimum(jnp.min(fmin_v), boff + P - 1) + zero16
                for q in range(k // 16):
                    lo_i = pl.multiple_of(kbase + q * 16, 16)
                    cur = idxg[pl.ds(lo_i, 16)]
                    slot = q * 16 + iota
                    idxg[pl.ds(lo_i, 16)] = jnp.where(
                        slot < base_v, cur, first)
                return 0

            jax.lax.fori_loop(0, g8, do_row, 0)

            def fire(rr, _):
                kbase = pl.multiple_of(rr * k, k)
                pltpu.async_copy(u_hbm.at[idxg.at[pl.ds(kbase, k)]],
                                 rowsg.at[pl.ds(kbase, k)], sem)
                return 0

            jax.lax.fori_loop(0, g8, fire, 0)

            def drain(rr, _):
                kbase = pl.multiple_of(rr * k, k)
                pltpu.make_async_copy(
                    u_hbm.at[idxg.at[pl.ds(kbase, k)]],
                    rowsg.at[pl.ds(kbase, k)], sem).wait()
                return 0

            jax.lax.fori_loop(0, g8, drain, 0)
            pltpu.sync_copy(
                rowsg,
                gout_hbm.at[pl.ds(pl.multiple_of(gbase * k, k), g8 * k)])
            return 0

        jax.lax.fori_loop(0, rpt // g8, do_group, 0)

    return body(d2_flat, lob, u)


def _mlp_body(g_ref, cnt_ref, cpos_ref, wb_ref, w2_ref, b2_ref, w3_ref,
              b3_ref, g3_ref, be3_ref, o_ref, *, k):
    C = cpos_ref[0]
    Sb = C.shape[0]
    H = jnp.dot(C, wb_ref[...], preferred_element_type=jnp.float32)
    rows = g_ref[0]
    D1 = rows.shape[-1]
    z1 = rows.reshape(Sb, k, D1) - H[:, None, :]
    h1 = jnp.maximum(z1, 0.0).reshape(Sb * k, D1)
    h2 = jnp.maximum(
        jnp.dot(h1, w2_ref[...], preferred_element_type=jnp.float32)
        + b2_ref[...], 0.0)
    h3 = jnp.maximum(
        jnp.dot(h2, w3_ref[...], preferred_element_type=jnp.float32)
        + b3_ref[...], 0.0)
    h3 = h3 * g3_ref[...] + be3_ref[...]
    D3 = h3.shape[-1]
    hv = h3.reshape(Sb, k, D3)
    m = k
    while m > 1:
        m //= 2
        hv = jnp.maximum(hv[:, :m, :], hv[:, m:2 * m, :])
    acc = hv.reshape(Sb, D3)
    o_ref[0] = jnp.where(cnt_ref[0] > 0, acc, 0.0)


def _sa_mlp(gathered, counts, cpos, params, k, sb, interpret=False):
    (w1, b1, g1, be1), (w2, b2, g2, be2), (w3, b3, g3, be3) = params
    F = w1.shape[0] - 3
    wb = w1[F:, :]
    w2f = g1[:, None] * w2
    b2f = b2 + be1 @ w2
    w3f = g2[:, None] * w3
    b3f = b3 + be2 @ w3
    B, S, _ = cpos.shape
    D1, D2 = w2.shape
    D3 = w3.shape[1]
    g3d = gathered.reshape(B, S * k, D1)
    return pl.pallas_call(
        functools.partial(_mlp_body, k=k),
        grid=(B, S // sb),
        in_specs=[
            pl.BlockSpec((1, sb * k, D1), lambda b, s: (b, s, 0)),
            pl.BlockSpec((1, sb, 1), lambda b, s: (b, s, 0)),
            pl.BlockSpec((1, sb, 3), lambda b, s: (b, s, 0)),
            pl.BlockSpec((3, D1), lambda b, s: (0, 0)),
            pl.BlockSpec((D1, D2), lambda b, s: (0, 0)),
            pl.BlockSpec((1, D2), lambda b, s: (0, 0)),
            pl.BlockSpec((D2, D3), lambda b, s: (0, 0)),
            pl.BlockSpec((1, D3), lambda b, s: (0, 0)),
            pl.BlockSpec((1, D3), lambda b, s: (0, 0)),
            pl.BlockSpec((1, D3), lambda b, s: (0, 0)),
        ],
        out_specs=pl.BlockSpec((1, sb, D3), lambda b, s: (b, s, 0)),
        out_shape=jax.ShapeDtypeStruct((B, S, D3), jnp.float32),
        interpret=interpret,
    )(g3d, counts, cpos, wb, w2f, b2f.reshape(1, D2), w3f,
      b3f.reshape(1, D3), g3.reshape(1, D3), be3.reshape(1, D3))


def _sa_module_sc(cpos, ppos_t, u, params, r, k, sb, interpret=False):
    B, S, _ = cpos.shape
    P = ppos_t.shape[2]
    D1 = u.shape[-1]
    d2i, lob, counts = _sa_select(cpos, ppos_t, r, k, sb, interpret)
    nrows = B * S
    gathered = _sc_gather(d2i.reshape(nrows * P), lob.reshape(nrows * 16),
                          u.reshape(B * P, D1), nrows, P, S, k, D1)
    return _sa_mlp(gathered, counts, cpos, params, k, sb, interpret)


def _sa_body(cpos_ref, ppt_ref, u_ref, wb_ref, w2_ref, b2_ref, w3_ref,
             b3_ref, g3_ref, be3_ref, o_ref, d2s_ref, *,
             k, r2_bits, pc, n_iter):
    C = cpos_ref[0]
    Pt = ppt_ref[0]
    Sb = C.shape[0]
    P = Pt.shape[1]
    dx = C[:, 0:1] - Pt[0:1, :]
    dy = C[:, 1:2] - Pt[1:2, :]
    dz = C[:, 2:3] - Pt[2:3, :]
    d2 = dx * dx + dy * dy + dz * dz
    d2i = jax.lax.bitcast_convert_type(d2, jnp.int32)
    d2s_ref[...] = d2i

    def bisect(_, carry):
        lo, hi = carry
        mid = (lo + hi) >> 1
        cnt = jnp.sum(jnp.where(d2i <= mid, 1.0, 0.0), axis=1, keepdims=True)
        good = cnt <= k
        return jnp.where(good, mid, lo), jnp.where(good, hi, mid)

    lo0 = jnp.full((Sb, 1), -1, jnp.int32)
    hi0 = jnp.full((Sb, 1), r2_bits + 1, jnp.int32)
    lo, _ = jax.lax.fori_loop(0, n_iter, bisect, (lo0, hi0))
    mask = d2i <= lo
    count = jnp.sum(jnp.where(mask, 1.0, 0.0), axis=1, keepdims=True)

    H = jnp.dot(C, wb_ref[...], preferred_element_type=jnp.float32)
    w2, b2 = w2_ref[...], b2_ref[...]
    w3, b3 = w3_ref[...], b3_ref[...]
    g3, be3 = g3_ref[...], be3_ref[...]
    D3 = w3.shape[1]

    def chunk(c, acc):
        base = pl.multiple_of(c * pc, pc)
        uc = u_ref[0, pl.ds(base, pc), :]
        z1 = uc[:, None, :] - H[None, :, :]
        h1 = jnp.maximum(z1, 0.0).reshape(pc * Sb, -1)
        h2 = jnp.maximum(
            jnp.dot(h1, w2, preferred_element_type=jnp.float32) + b2, 0.0)
        h3 = jnp.maximum(
            jnp.dot(h2, w3, preferred_element_type=jnp.float32) + b3, 0.0)
        h3 = h3 * g3 + be3
        mc = d2s_ref[:, pl.ds(base, pc)] <= lo
        pen = jnp.where(mc, 0.0, -1e30)
        for j in range(pc):
            hv = h3[j * Sb:(j + 1) * Sb, :] + pen[:, j:j + 1]
            acc = jnp.maximum(acc, hv)
        return acc

    acc0 = jnp.full((Sb, D3), -1e30, jnp.float32)
    acc = jax.lax.fori_loop(0, P // pc, chunk, acc0)
    o_ref[0] = jnp.where(count > 0, acc, 0.0)


def _sa_dense(cpos, ppos_t, u, params, r, k, sb, pc, interpret=False):
    (w1, b1, g1, be1), (w2, b2, g2, be2), (w3, b3, g3, be3) = params
    F = w1.shape[0] - 3
    wb = w1[F:, :]
    w2f = g1[:, None] * w2
    b2f = b2 + be1 @ w2
    w3f = g2[:, None] * w3
    b3f = b3 + be2 @ w3
    B, S, _ = cpos.shape
    P = ppos_t.shape[2]
    D1, D2 = w2.shape
    D3 = w3.shape[1]
    r2_bits = int(np.float32(r * r).view(np.int32))
    grid = (B, S // sb)
    return pl.pallas_call(
        functools.partial(_sa_body, k=k, r2_bits=r2_bits, pc=pc, n_iter=31),
        grid=grid,
        in_specs=[
            pl.BlockSpec((1, sb, 3), lambda b, s: (b, s, 0)),
            pl.BlockSpec((1, 3, P), lambda b, s: (b, 0, 0)),
            pl.BlockSpec((1, P, D1), lambda b, s: (b, 0, 0)),
            pl.BlockSpec((3, D1), lambda b, s: (0, 0)),
            pl.BlockSpec((D1, D2), lambda b, s: (0, 0)),
            pl.BlockSpec((1, D2), lambda b, s: (0, 0)),
            pl.BlockSpec((D2, D3), lambda b, s: (0, 0)),
            pl.BlockSpec((1, D3), lambda b, s: (0, 0)),
            pl.BlockSpec((1, D3), lambda b, s: (0, 0)),
            pl.BlockSpec((1, D3), lambda b, s: (0, 0)),
        ],
        out_specs=pl.BlockSpec((1, sb, D3), lambda b, s: (b, s, 0)),
        out_shape=jax.ShapeDtypeStruct((B, S, D3), jnp.float32),
        scratch_shapes=[pltpu.VMEM((sb, P), jnp.int32)],
        interpret=interpret,
    )(cpos, ppos_t, u, wb, w2f, b2f.reshape(1, D2), w3f,
      b3f.reshape(1, D3), g3.reshape(1, D3), be3.reshape(1, D3))


def _pointnet2(x_0, pos_0, x_1, pos_1, p1, p2, *, k, r1, r2, interpret=False):
    B = 2
    P = x_0.shape[0] // B
    F = x_0.shape[-1]
    S1, S2 = P // 2, P // 8
    d1a = p1[0][0].shape[1]
    d1b = p2[0][0].shape[1]

    pos0 = pos_0.reshape(B, P, 3)
    pos1 = pos_1.reshape(B, P, 3)

    cx0, cy0, cz0 = _fps(pos0[:, :, 0], pos0[:, :, 1], pos0[:, :, 2], S1,
                         interpret)
    cx1, cy1, cz1 = _fps(cx0, cy0, cz0, S2, interpret)
    c0 = jnp.stack([cx0, cy0, cz0], axis=-1)
    c0_t = jnp.stack([cx0, cy0, cz0], axis=1)
    c1 = jnp.stack([cx1, cy1, cz1], axis=-1)

    pos0_t = jnp.transpose(pos0, (0, 2, 1))
    pos1_t = jnp.transpose(pos1, (0, 2, 1))

    sb1 = min(128, S1)
    sb2 = min(128, S2)
    pc1 = min(128, P)
    pc2 = min(128, S1)

    if interpret:
        def sa(cpos, ppos_t, u, params, r, sb, pc):
            return _sa_dense(cpos, ppos_t, u, params, r, k, sb, pc, True)
    else:
        def sa(cpos, ppos_t, u, params, r, sb, pc):
            return _sa_module_sc(cpos, ppos_t, u, params, r, k, sb)

    u1 = _lin1(x_0, pos_0, p1[0][0][:F], p1[0][0][F:], p1[0][1], interpret)
    x1_0 = sa(c0, pos0_t, u1.reshape(B, P, d1a), p1, r1, sb1, pc1)

    u3 = _lin1(x_1, pos_1, p1[0][0][:F], p1[0][0][F:], p1[0][1], interpret)
    x1_1 = sa(c0, pos1_t, u3.reshape(B, P, d1a), p1, r1, sb1, pc1)

    Fm = x1_0.shape[-1]
    u2 = _lin1(x1_0.reshape(B * S1, Fm), c0.reshape(B * S1, 3),
               p2[0][0][:Fm], p2[0][0][Fm:], p2[0][1], interpret)
    x2_0 = sa(c1, c0_t, u2.reshape(B, S1, d1b), p2, r2, min(64, S2), pc2)

    u4 = _lin1(x1_1.reshape(B * S1, Fm), c0.reshape(B * S1, 3),
               p1[0][0][:Fm], p1[0][0][Fm:], p1[0][1], interpret)
    x2_1 = sa(c1, c0_t, u4.reshape(B, S1, d1a), p1, r1, min(64, S2), pc2)

    return x2_0, c1, x2_1


def kernel(x_0, pos_0, batch_0, x_1, pos_1, batch_1, p1, p2):
    B = 2
    P = x_0.shape[0] // B
    S2 = P // 8
    x2_0, c1, x2_1 = _pointnet2(x_0, pos_0, x_1, pos_1, p1, p2,
                                k=64, r1=0.2, r2=0.4)
    batch_c1 = batch_0.reshape(B, P)[:, :S2].reshape(-1)
    sa2_out_0 = (x2_0.reshape(-1, x2_0.shape[-1]), c1.reshape(-1, 3), batch_c1)
    sa2_out_1 = (x2_1.reshape(-1, x2_1.shape[-1]), c1.reshape(-1, 3), batch_c1)
    return (sa2_out_0, sa2_out_1)
